# Initial kernel scaffold; baseline (speedup 1.0000x reference)
import jax
import jax.numpy as jnp
from jax.experimental import pallas as pl


def kernel(species, coordinates):
    raise NotImplementedError("write your pallas kernel here")


# speedup vs baseline: 12641.5847x; 12641.5847x over previous
import functools
import math

import jax
import jax.numpy as jnp
import numpy as np
from jax import lax
from jax.experimental import pallas as pl
from jax.experimental.pallas import tpu as pltpu
from jax.experimental.pallas import tpu_sc as plsc

_RCR = 5.2
_RCA = 3.5
_ETA_R = 16.0
_ETA_A = 8.0
_SHFR = np.array([0.9, 1.16875, 1.4375, 1.70625, 1.975, 2.24375, 2.5125,
                  2.78125, 3.05, 3.31875, 3.5875, 3.85625, 4.125, 4.39375,
                  4.6625, 4.93125], dtype=np.float32)
_SHFA = np.array([0.9, 1.55, 2.2, 2.85], dtype=np.float32)
_SHFZ = np.array([0.19634954, 0.58904862, 0.9817477, 1.3744468, 1.7671459,
                  2.1598449, 2.552544, 2.9452431], dtype=np.float32)
_NSP = 4
_RAD_LEN = 64
_ANG_LEN = 320
_NCAP = 32
_L = 16
_PAD = 16

_COS_COEF = (1.0000000000, -0.499999995, 4.16666419e-02, -1.38884324e-03,
             2.47637706e-05, -2.61150239e-07)


def _vrsqrt(x):
    i = plsc.bitcast(x, jnp.int32)
    y = plsc.bitcast(jnp.int32(0x5F3759DF) - (i >> 1), jnp.float32)
    for _ in range(3):
        y = y * (1.5 - 0.5 * x * y * y)
    return y


def _fc_vec(d, rc):
    y = d * (math.pi / (2.0 * rc))
    u = y * y
    p = jnp.full((_L,), _COS_COEF[-1], jnp.float32)
    for c in _COS_COEF[-2::-1]:
        p = p * u + c
    return p * p


def _sget(ref, i):
    return ref[pl.ds(i, _L)][0]


def _sget2(ref, i, j):
    return ref[i, pl.ds(j, _L)][0]


def _aev_body(x_hbm, y_hbm, z_hbm, sp_hbm, shfr_hbm, cz_hbm, sz_hbm, sa_hbm,
              out_r_hbm, out_a_hbm,
              xv, yv, zv, spv, shfr_v, cz_v, sz_v, sa_v,
              rad_v, ang_v, nb_d, nb_j,
              a_d, a_fc, a_dx, a_dy, a_dz, a_sp, cnt_v):
    num_mols = x_hbm.shape[0]
    natoms = x_hbm.shape[1]
    nchunks = natoms // _L
    mols_per_w = num_mols // 32

    wid = lax.axis_index("s") * 2 + lax.axis_index("c")

    pltpu.sync_copy(shfr_hbm, shfr_v)
    pltpu.sync_copy(cz_hbm, cz_v)
    pltpu.sync_copy(sz_hbm, sz_v)
    pltpu.sync_copy(sa_hbm, sa_v)

    zero16 = jnp.zeros((_L,), jnp.float32)
    lane_iota = lax.iota(jnp.int32, _L)
    rcr2 = jnp.float32(_RCR * _RCR)
    rca = jnp.float32(_RCA)

    def mol_body(m, carry):
        mol = wid * mols_per_w + m
        for src, dst in ((x_hbm, xv), (y_hbm, yv), (z_hbm, zv)):
            pltpu.sync_copy(src.at[mol], dst.at[pl.ds(0, natoms)])
        pltpu.sync_copy(sp_hbm.at[mol], spv.at[pl.ds(0, natoms)])

        def zrow(i, c2):
            for c in range(_RAD_LEN // _L):
                rad_v[i, pl.ds(c * _L, _L)] = zero16
            for c in range(_ANG_LEN // _L):
                ang_v[i, pl.ds(c * _L, _L)] = zero16
            return c2
        lax.fori_loop(0, natoms, zrow, 0)

        def sweep(i, c2):
            xi = _sget(xv, i)
            yi = _sget(yv, i)
            zi = _sget(zv, i)
            row_i = jnp.full((_L,), i, jnp.int32)

            def chunk(c, cnt):
                base = c * _L
                xs = xv[pl.ds(base, _L)]
                ys = yv[pl.ds(base, _L)]
                zs = zv[pl.ds(base, _L)]
                dx = xi - xs
                dy = yi - ys
                dz = zi - zs
                d2 = dx * dx + dy * dy + dz * dz
                jids = lane_iota + base
                msk = (d2 <= rcr2) & (jids != i)
                r = _vrsqrt(jnp.maximum(d2, jnp.float32(1e-12)))
                d = d2 * r
                pos = plsc.cumsum(jnp.where(msk, 1, 0))
                col = jnp.minimum(cnt + pos - 1, _NCAP - 1)
                idx = [row_i, col]
                plsc.store_scatter(nb_d, idx, d, mask=msk)
                plsc.store_scatter(nb_j, idx, jids, mask=msk)
                return cnt + jnp.max(pos)

            cnt = lax.fori_loop(0, nchunks, chunk, jnp.int32(0))
            cnt_v[i] = jnp.minimum(cnt, _NCAP)
            return c2
        lax.fori_loop(0, natoms, sweep, 0)

        def rad_atom(i, c2):
            n = cnt_v[i]

            def rpair(j, c3):
                d = _sget2(nb_d, i, j)
                dv = jnp.full((_L,), d, jnp.float32)
                fc = _fc_vec(dv, _RCR)
                s = _sget(spv, _sget2(nb_j, i, j))
                t = dv - shfr_v[...]
                rt = (0.25 * fc) * jnp.exp(t * t * (-_ETA_R))
                rad_v[i, pl.ds(s * _L, _L)] += rt
                return c3
            lax.fori_loop(0, n, rpair, 0)
            return c2
        lax.fori_loop(0, natoms, rad_atom, 0)

        def ang_atom(i, c2):
            n = cnt_v[i]
            xi = jnp.full((_L,), _sget(xv, i), jnp.float32)
            yi = jnp.full((_L,), _sget(yv, i), jnp.float32)
            zi = jnp.full((_L,), _sget(zv, i), jnp.float32)

            def asel(c, na):
                base = c * _L
                lanes = lane_iota + base
                d = nb_d[i, pl.ds(base, _L)]
                jj = nb_j[i, pl.ds(base, _L)]
                msk = (lanes < n) & (d <= rca)
                fca = _fc_vec(d, _RCA)
                jc = jnp.where(msk, jj, 0)
                dx = xi - plsc.load_gather(xv, [jc])
                dy = yi - plsc.load_gather(yv, [jc])
                dz = zi - plsc.load_gather(zv, [jc])
                spj = plsc.load_gather(spv, [jc])
                pos = plsc.cumsum(jnp.where(msk, 1, 0))
                col = [na + pos - 1]
                plsc.store_scatter(a_d, col, d, mask=msk)
                plsc.store_scatter(a_fc, col, fca, mask=msk)
                plsc.store_scatter(a_dx, col, dx, mask=msk)
                plsc.store_scatter(a_dy, col, dy, mask=msk)
                plsc.store_scatter(a_dz, col, dz, mask=msk)
                plsc.store_scatter(a_sp, col, spj, mask=msk)
                return na + jnp.max(pos)

            na = lax.fori_loop(0, _NCAP // _L, asel, jnp.int32(0))

            def k1loop(k1, c3):
                d1 = _sget(a_d, k1)
                f1 = _sget(a_fc, k1)
                x1 = _sget(a_dx, k1)
                y1 = _sget(a_dy, k1)
                z1 = _sget(a_dz, k1)
                s1 = _sget(a_sp, k1)

                def k2loop(k2, c4):
                    d2s = _sget(a_d, k2)
                    dot = (x1 * _sget(a_dx, k2) + y1 * _sget(a_dy, k2)
                           + z1 * _sget(a_dz, k2))
                    s2 = _sget(a_sp, k2)
                    pn = jnp.maximum(d1 * d2s, jnp.float32(1e-8))
                    pnv = jnp.full((_L,), pn, jnp.float32)
                    cosv = jnp.full((_L,), jnp.float32(0.95) * dot,
                                    jnp.float32) / pnv
                    sin2v = jnp.maximum(1.0 - cosv * cosv,
                                        jnp.float32(1e-20))
                    sinv = sin2v * _vrsqrt(sin2v)
                    davg = jnp.full((_L,), 0.5 * (d1 + d2s), jnp.float32)
                    g = 2.0 * f1 * _sget(a_fc, k2)
                    sa_lo = jnp.minimum(s1, s2)
                    sa_hi = jnp.maximum(s1, s2)
                    p = sa_lo * _NSP + sa_hi - ((sa_lo * (sa_lo + 1)) >> 1)
                    cb = p * 32
                    for c in range(2):
                        sl = pl.ds(c * _L, _L)
                        cosd = cosv * cz_v[sl] + sinv * sz_v[sl]
                        x = 0.5 + 0.5 * cosd
                        for _ in range(5):
                            x = x * x
                        ta = davg - sa_v[sl]
                        term = (x * g) * jnp.exp(ta * ta * (-_ETA_A))
                        ang_v[i, pl.ds(cb + c * _L, _L)] += term
                    return c4
                lax.fori_loop(k1 + 1, na, k2loop, 0)
                return c3
            lax.fori_loop(0, na, k1loop, 0)
            return c2
        lax.fori_loop(0, natoms, ang_atom, 0)

        pltpu.sync_copy(rad_v, out_r_hbm.at[mol])
        pltpu.sync_copy(ang_v, out_a_hbm.at[mol])
        return carry

    lax.fori_loop(0, mols_per_w, mol_body, 0)


@jax.jit
def _aev_sc(xc, yc, zc, sp):
    num_mols, natoms = xc.shape

    zq = np.tile(_SHFZ, 4)
    sq = np.repeat(_SHFA, 8)
    cz = np.cos(zq).astype(np.float32)
    sz = np.sin(zq).astype(np.float32)
    sa = sq.astype(np.float32)

    kern = pl.kernel(
        _aev_body,
        out_type=(
            jax.ShapeDtypeStruct((num_mols, natoms, _RAD_LEN), jnp.float32),
            jax.ShapeDtypeStruct((num_mols, natoms, _ANG_LEN), jnp.float32),
        ),
        mesh=plsc.VectorSubcoreMesh(core_axis_name="c", subcore_axis_name="s"),
        compiler_params=pltpu.CompilerParams(needs_layout_passes=False),
        scratch_types=[
            pltpu.VMEM((natoms + _PAD,), jnp.float32),
            pltpu.VMEM((natoms + _PAD,), jnp.float32),
            pltpu.VMEM((natoms + _PAD,), jnp.float32),
            pltpu.VMEM((natoms + _PAD,), jnp.int32),
            pltpu.VMEM((_L,), jnp.float32),
            pltpu.VMEM((2 * _L,), jnp.float32),
            pltpu.VMEM((2 * _L,), jnp.float32),
            pltpu.VMEM((2 * _L,), jnp.float32),
            pltpu.VMEM((natoms, _RAD_LEN), jnp.float32),
            pltpu.VMEM((natoms, _ANG_LEN), jnp.float32),
            pltpu.VMEM((natoms, _NCAP + _PAD), jnp.float32),
            pltpu.VMEM((natoms, _NCAP + _PAD), jnp.int32),
            pltpu.VMEM((_NCAP + _PAD,), jnp.float32),
            pltpu.VMEM((_NCAP + _PAD,), jnp.float32),
            pltpu.VMEM((_NCAP + _PAD,), jnp.float32),
            pltpu.VMEM((_NCAP + _PAD,), jnp.float32),
            pltpu.VMEM((_NCAP + _PAD,), jnp.float32),
            pltpu.VMEM((_NCAP + _PAD,), jnp.int32),
            pltpu.SMEM((natoms,), jnp.int32),
        ],
    )
    return kern(xc, yc, zc, sp,
                jnp.asarray(_SHFR), jnp.asarray(cz), jnp.asarray(sz),
                jnp.asarray(sa))


def kernel(species, coordinates):
    c = coordinates.astype(jnp.float32)
    sp = species.astype(jnp.int32)
    out_r, out_a = _aev_sc(c[:, :, 0], c[:, :, 1], c[:, :, 2], sp)
    return jnp.concatenate([out_r, out_a], axis=-1)
